# baseline (speedup 1.0000x reference)
import functools

import jax
import jax.numpy as jnp
from jax import lax
from jax.experimental import pallas as pl
from jax.experimental.pallas import tpu as pltpu
from jax.experimental.pallas import tpu_sc as plsc

_H = 224
_W = 224
_HW = _H * _W
_N = 2
_C = 384
_TOT = _N * _HW
_NC = 2
_NS = 16
_NW = _NC * _NS
_SPW = _TOT // _NW
_SS = 32
_NCH = _SPW // _SS


def _prep_body(gx_ref, gy_ref, iv_ref, w_ref):
    gx = gx_ref[...]
    gy = gy_ref[...]
    ix = ((gx + 1.0) * _W - 1.0) * 0.5
    iy = ((gy + 1.0) * _H - 1.0) * 0.5
    x0f = jnp.clip(jnp.floor(ix), -2.0, float(_W))
    y0f = jnp.clip(jnp.floor(iy), -2.0, float(_H))
    wx1 = ix - x0f
    wx0 = 1.0 - wx1
    wy1 = iy - y0f
    wy0 = 1.0 - wy1
    x0 = x0f.astype(jnp.int32)
    y0 = y0f.astype(jnp.int32)
    inx = ((x0 >= 0) & (x0 <= _W - 2)).astype(jnp.float32)
    iny = ((y0 >= 0) & (y0 <= _H - 2)).astype(jnp.float32)
    wl = wx0 * inx + wx1 * (x0 == -1)
    wr = wx1 * inx + wx0 * (x0 == _W - 1)
    wt = wy0 * iny + wy1 * (y0 == -1)
    wb = wy1 * iny + wy0 * (y0 == _H - 1)
    xb = jnp.clip(x0, 0, _W - 2)
    yb = jnp.clip(y0, 0, _H - 2)
    base = jnp.arange(_N, dtype=jnp.int32).reshape(_N, 1) * _HW
    iv_ref[...] = base + yb * _W + xb
    w_ref[...] = jnp.stack(
        [wt * wl, wt * wr, wb * wl, wb * wr], axis=0
    )


def _prep(gx, gy):
    return pl.pallas_call(
        _prep_body,
        out_shape=(
            jax.ShapeDtypeStruct((_N, _HW), jnp.int32),
            jax.ShapeDtypeStruct((4, _N, _HW), jnp.float32),
        ),
    )(gx, gy)


_mesh = plsc.VectorSubcoreMesh(core_axis_name="c", subcore_axis_name="s")


@functools.partial(
    pl.kernel,
    out_type=jax.ShapeDtypeStruct((_TOT, _C), jnp.float32),
    mesh=_mesh,
    compiler_params=pltpu.CompilerParams(needs_layout_passes=False),
    scratch_types=[
        (pltpu.VMEM((_SS,), jnp.int32),
         pltpu.VMEM((_SS,), jnp.int32)),
        (pltpu.VMEM((4, _SS), jnp.float32),
         pltpu.VMEM((4, _SS), jnp.float32)),
        pltpu.VMEM((2, 4, _SS), jnp.int32),
        pltpu.VMEM((2, 4, _SS, _C), jnp.float32),
        pltpu.VMEM((2, _SS, _C), jnp.float32),
        (pltpu.SemaphoreType.DMA, pltpu.SemaphoreType.DMA),
        (pltpu.SemaphoreType.DMA, pltpu.SemaphoreType.DMA),
        (pltpu.SemaphoreType.DMA, pltpu.SemaphoreType.DMA),
        (pltpu.SemaphoreType.DMA, pltpu.SemaphoreType.DMA),
    ],
)
def _sc_rowgather(
    xt_hbm, iv_hbm, w_hbm, out_hbm,
    ivc_v, wc_v, idx_v, g_v, out_v, sem_iv, sem_w, sem_g, sem_out,
):
    wid = lax.axis_index("s") * _NC + lax.axis_index("c")
    gbase = wid * _SPW

    iota = lax.iota(jnp.int32, 16)

    def iv_copy(c, buf):
        return pltpu.make_async_copy(iv_hbm.at[wid, c], ivc_v[buf], sem_iv[buf])

    def w_copy(c, buf):
        return pltpu.make_async_copy(w_hbm.at[wid, c], wc_v[buf], sem_w[buf])

    def gather_copy(t, buf):
        return pltpu.make_async_copy(
            xt_hbm.at[idx_v.at[buf, t]], g_v.at[buf, t], sem_g[buf]
        )

    def out_copy(c, buf):
        return pltpu.make_async_copy(
            out_v.at[buf], out_hbm.at[pl.ds(gbase + c * _SS, _SS)], sem_out[buf]
        )

    def fire(c, buf):
        iv_copy(c, buf).wait()
        for g16 in (0, 1):
            sl = pl.ds(g16 * 16, 16)
            iv = ivc_v[buf][sl]
            idx_v[buf, 0, sl] = iv
            idx_v[buf, 1, sl] = iv + 1
            idx_v[buf, 2, sl] = iv + _W
            idx_v[buf, 3, sl] = iv + (_W + 1)
        for t in range(4):
            gather_copy(t, buf).start()

    def blend(c, buf):
        for t in range(4):
            gather_copy(t, buf).wait()
        for g16 in (0, 1):
            sl = pl.ds(g16 * 16, 16)
            w00 = wc_v[buf][0, sl]
            w01 = wc_v[buf][1, sl]
            w10 = wc_v[buf][2, sl]
            w11 = wc_v[buf][3, sl]
            srow = iota + (g16 * 16)

            @plsc.parallel_loop(0, _C, step=1, unroll=4)
            def ch_body(ch):
                chv = jnp.full((16,), ch, dtype=jnp.int32)
                g00 = plsc.load_gather(g_v.at[buf, 0], [srow, chv])
                g01 = plsc.load_gather(g_v.at[buf, 1], [srow, chv])
                g10 = plsc.load_gather(g_v.at[buf, 2], [srow, chv])
                g11 = plsc.load_gather(g_v.at[buf, 3], [srow, chv])
                acc = g00 * w00 + g01 * w01 + g10 * w10 + g11 * w11
                plsc.store_scatter(out_v.at[buf], [srow, chv], acc)

        out_copy(c, buf).start()

    iv_copy(0, 0).start()
    iv_copy(1, 1).start()
    w_copy(0, 0).start()
    w_copy(1, 1).start()

    def chunk2_body(cc, _):
        for buf in (0, 1):
            c = cc * 2 + buf
            fire(c, buf)

            @pl.when(c >= 1)
            def _():
                pbuf = 1 - buf
                w_copy(c - 1, pbuf).wait()

                @pl.when(c >= 3)
                def _():
                    out_copy(c - 3, pbuf).wait()

                blend(c - 1, pbuf)

                @pl.when(c + 1 < _NCH)
                def _():
                    w_copy(c + 1, pbuf).start()

            @pl.when(c + 2 < _NCH)
            def _():
                iv_copy(c + 2, buf).start()

        return 0

    lax.fori_loop(0, _NCH // 2, chunk2_body, 0)

    last = _NCH - 1
    lbuf = last & 1
    w_copy(last, lbuf).wait()
    out_copy(last - 2, lbuf).wait()
    blend(last, lbuf)
    out_copy(last - 1, 1 - lbuf).wait()
    out_copy(last, lbuf).wait()


def kernel(x, grid):
    gx = grid[..., 0].reshape(_N, _HW)
    gy = grid[..., 1].reshape(_N, _HW)
    iv, w4 = _prep(gx, gy)
    iv = iv.reshape(_NW, _NCH, _SS)
    w4 = w4.reshape(4, _NW, _NCH, _SS).transpose(1, 2, 0, 3)
    xt = x.transpose(0, 2, 3, 1).reshape(_TOT, _C)
    outt = _sc_rowgather(xt, iv, w4)
    return outt.reshape(_N, _H, _W, _C).transpose(0, 3, 1, 2)


# speedup vs baseline: 2.4024x; 2.4024x over previous
import functools

import jax
import jax.numpy as jnp
from jax import lax
from jax.experimental import pallas as pl
from jax.experimental.pallas import tpu as pltpu
from jax.experimental.pallas import tpu_sc as plsc

_H = 224
_W = 224
_HW = _H * _W
_N = 2
_C = 384
_CH = _C // 2
_NC = 2
_NS = 16
_NW = _NC * _NS
_PPW = _N * _CH // _NW
_RC = 2
_S = _RC * _W
_NCHUNK = _H // _RC
_CROWS = 16
_MASK_HI = -65536


def _pack_body(x_ref, xp_ref):
    for j in range(4):
        a = x_ref[2 * j]
        b = x_ref[2 * j + 1]
        au = lax.bitcast_convert_type(a.astype(jnp.bfloat16), jnp.uint16)
        bu = lax.bitcast_convert_type(b.astype(jnp.bfloat16), jnp.uint16)
        word = (au.astype(jnp.uint32) << 16) | bu.astype(jnp.uint32)
        w = lax.bitcast_convert_type(word, jnp.int32)
        halves = jnp.stack(
            [w[:, :128], jnp.pad(w[:, 128:], ((0, 0), (0, 32)))], axis=1
        )
        xp_ref[j] = halves.reshape(2 * _H, 128)


def _pack(x):
    return pl.pallas_call(
        _pack_body,
        grid=(_N * _C // 8,),
        in_specs=[pl.BlockSpec((8, _H, _W), lambda p: (p, 0, 0))],
        out_specs=pl.BlockSpec((4, 2 * _H, 128), lambda p: (p, 0, 0)),
        out_shape=jax.ShapeDtypeStruct((_N * _CH, 2 * _H, 128), jnp.int32),
    )(x.reshape(_N * _C, _H, _W))


def _prep_body(gx_ref, gy_ref, iw_ref):
    gx = gx_ref[...]
    gy = gy_ref[...]
    ix = ((gx + 1.0) * _W - 1.0) * 0.5
    iy = ((gy + 1.0) * _H - 1.0) * 0.5
    x0f = jnp.clip(jnp.floor(ix), -2.0, float(_W))
    y0f = jnp.clip(jnp.floor(iy), -2.0, float(_H))
    wx1 = ix - x0f
    wx0 = 1.0 - wx1
    wy1 = iy - y0f
    wy0 = 1.0 - wy1
    x0 = x0f.astype(jnp.int32)
    y0 = y0f.astype(jnp.int32)
    inx = ((x0 >= 0) & (x0 <= _W - 2)).astype(jnp.float32)
    iny = ((y0 >= 0) & (y0 <= _H - 2)).astype(jnp.float32)
    wl = wx0 * inx + wx1 * (x0 == -1)
    wr = wx1 * inx + wx0 * (x0 == _W - 1)
    wt = wy0 * iny + wy1 * (y0 == -1)
    wb = wy1 * iny + wy0 * (y0 == _H - 1)
    xb = jnp.clip(x0, 0, _W - 2)
    yb = jnp.clip(y0, 0, _H - 2)
    iv_f = lax.bitcast_convert_type((yb << 8) | xb, jnp.float32)
    rows = [iv_f, wt * wl, wt * wr, wb * wl, wb * wr]
    stacked = jnp.stack(
        [r.reshape(_N, _NCHUNK, _RC, _W) for r in rows], axis=2
    ).reshape(_N, _NCHUNK, 5 * _RC, _W)
    padded = jnp.pad(
        stacked, ((0, 0), (0, 0), (0, _CROWS - 5 * _RC), (0, 256 - _W))
    )
    iw_ref[...] = padded.reshape(_N, _NCHUNK * _CROWS, 256)


def _prep(gx, gy):
    return pl.pallas_call(
        _prep_body,
        out_shape=jax.ShapeDtypeStruct((_N, _NCHUNK * _CROWS, 256), jnp.float32),
    )(gx, gy)


_mesh = plsc.VectorSubcoreMesh(core_axis_name="c", subcore_axis_name="s")


@functools.partial(
    pl.kernel,
    out_type=jax.ShapeDtypeStruct((_N, _C, _H, _W), jnp.float32),
    mesh=_mesh,
    compiler_params=pltpu.CompilerParams(needs_layout_passes=False),
    scratch_types=[
        pltpu.VMEM((2 * _H, 128), jnp.int32),
        pltpu.VMEM((2 * _H, 128), jnp.int32),
        (pltpu.VMEM((_CROWS, 256), jnp.float32),
         pltpu.VMEM((_CROWS, 256), jnp.float32)),
        pltpu.VMEM((2, 4, _RC, _W), jnp.float32),
        pltpu.SemaphoreType.DMA,
        (pltpu.SemaphoreType.DMA, pltpu.SemaphoreType.DMA),
        (pltpu.SemaphoreType.DMA, pltpu.SemaphoreType.DMA),
    ],
)
def _sc_sample(
    xp_hbm, iw_hbm, out_hbm,
    pp0_v, pp1_v, iw_v, out_v, sem_pl, sem_iw, sem_out,
):
    wid = lax.axis_index("s") * _NC + lax.axis_index("c")
    base_pp = wid * _PPW
    b = base_pp // _CH
    base_q = base_pp - b * _CH

    def iw_copy(c, buf):
        return pltpu.make_async_copy(
            iw_hbm.at[b, pl.ds(c * _CROWS, _CROWS)], iw_v[buf], sem_iw[buf]
        )

    def out_copy(buf, ch, c):
        return pltpu.make_async_copy(
            out_v.at[buf],
            out_hbm.at[b, pl.ds(ch, 4), pl.ds(c * _RC, _RC)],
            sem_out[buf],
        )

    def pair_body(pp, _):
        q0 = base_pp + 2 * pp
        ch0 = 2 * (base_q + 2 * pp)
        cp0 = pltpu.async_copy(xp_hbm.at[q0], pp0_v, sem_pl)
        cp1 = pltpu.async_copy(xp_hbm.at[q0 + 1], pp1_v, sem_pl)
        iw_copy(0, 0).start()
        iw_copy(1, 1).start()
        cp0.wait()
        cp1.wait()

        def chunk2_body(cc, _):
            for buf in (0, 1):
                c = cc * 2 + buf
                iw_copy(c, buf).wait()

                @pl.when(c >= 2)
                def _():
                    out_copy(buf, ch0, c).wait()

                iwb = iw_v[buf]

                @plsc.parallel_loop(0, _W, step=16, unroll=2)
                def vec_body(x0):
                    for r in range(_RC):
                        iv = plsc.bitcast(iwb[r, pl.ds(x0, 16)], jnp.int32)
                        w00 = iwb[_RC + r, pl.ds(x0, 16)]
                        w01 = iwb[2 * _RC + r, pl.ds(x0, 16)]
                        w10 = iwb[3 * _RC + r, pl.ds(x0, 16)]
                        w11 = iwb[4 * _RC + r, pl.ds(x0, 16)]
                        iv1 = iv + 1
                        iv2 = iv + 256
                        iv3 = iv + 257
                        taps = [(t >> 7, t & 127) for t in (iv, iv1, iv2, iv3)]
                        for k, ppv in ((0, pp0_v), (1, pp1_v)):
                            g00 = plsc.load_gather(ppv, list(taps[0]))
                            g01 = plsc.load_gather(ppv, list(taps[1]))
                            g10 = plsc.load_gather(ppv, list(taps[2]))
                            g11 = plsc.load_gather(ppv, list(taps[3]))
                            acc_a = (
                                plsc.bitcast(g00 & _MASK_HI, jnp.float32) * w00
                                + plsc.bitcast(g01 & _MASK_HI, jnp.float32) * w01
                                + plsc.bitcast(g10 & _MASK_HI, jnp.float32) * w10
                                + plsc.bitcast(g11 & _MASK_HI, jnp.float32) * w11
                            )
                            acc_b = (
                                plsc.bitcast(g00 << 16, jnp.float32) * w00
                                + plsc.bitcast(g01 << 16, jnp.float32) * w01
                                + plsc.bitcast(g10 << 16, jnp.float32) * w10
                                + plsc.bitcast(g11 << 16, jnp.float32) * w11
                            )
                            out_v[buf, 2 * k, r, pl.ds(x0, 16)] = acc_a
                            out_v[buf, 2 * k + 1, r, pl.ds(x0, 16)] = acc_b

                @pl.when(c + 2 < _NCHUNK)
                def _():
                    iw_copy(c + 2, buf).start()

                out_copy(buf, ch0, c).start()
            return 0

        lax.fori_loop(0, _NCHUNK // 2, chunk2_body, 0)
        for buf in (0, 1):
            out_copy(buf, ch0, 0).wait()
        return 0

    lax.fori_loop(0, _PPW // 2, pair_body, 0)


def kernel(x, grid):
    xp = _pack(x)
    gx = grid[..., 0].reshape(_N, _HW)
    gy = grid[..., 1].reshape(_N, _HW)
    iw = _prep(gx, gy)
    return _sc_sample(xp, iw)


# speedup vs baseline: 2.4055x; 1.0013x over previous
import functools

import jax
import jax.numpy as jnp
from jax import lax
from jax.experimental import pallas as pl
from jax.experimental.pallas import tpu as pltpu
from jax.experimental.pallas import tpu_sc as plsc

_H = 224
_W = 224
_HW = _H * _W
_N = 2
_C = 384
_CH = _C // 2
_NC = 2
_NS = 16
_NW = _NC * _NS
_PPW = _N * _CH // _NW
_RC = 2
_S = _RC * _W
_NCHUNK = _H // _RC
_CROWS = 16
_MASK_HI = -65536


def _pack_body(x_ref, xp_ref):
    for j in range(4):
        a = x_ref[2 * j]
        b = x_ref[2 * j + 1]
        au = lax.bitcast_convert_type(a.astype(jnp.bfloat16), jnp.uint16)
        bu = lax.bitcast_convert_type(b.astype(jnp.bfloat16), jnp.uint16)
        word = (au.astype(jnp.uint32) << 16) | bu.astype(jnp.uint32)
        w = lax.bitcast_convert_type(word, jnp.int32)
        halves = jnp.stack(
            [w[:, :128], jnp.pad(w[:, 128:], ((0, 0), (0, 32)))], axis=1
        )
        xp_ref[j] = halves.reshape(2 * _H, 128)


def _pack(x):
    return pl.pallas_call(
        _pack_body,
        grid=(_N * _C // 8,),
        in_specs=[pl.BlockSpec((8, _H, _W), lambda p: (p, 0, 0))],
        out_specs=pl.BlockSpec((4, 2 * _H, 128), lambda p: (p, 0, 0)),
        out_shape=jax.ShapeDtypeStruct((_N * _CH, 2 * _H, 128), jnp.int32),
    )(x.reshape(_N * _C, _H, _W))


def _prep_body(gx_ref, gy_ref, iw_ref):
    gx = gx_ref[...]
    gy = gy_ref[...]
    ix = ((gx + 1.0) * _W - 1.0) * 0.5
    iy = ((gy + 1.0) * _H - 1.0) * 0.5
    x0f = jnp.clip(jnp.floor(ix), -2.0, float(_W))
    y0f = jnp.clip(jnp.floor(iy), -2.0, float(_H))
    wx1 = ix - x0f
    wx0 = 1.0 - wx1
    wy1 = iy - y0f
    wy0 = 1.0 - wy1
    x0 = x0f.astype(jnp.int32)
    y0 = y0f.astype(jnp.int32)
    inx = ((x0 >= 0) & (x0 <= _W - 2)).astype(jnp.float32)
    iny = ((y0 >= 0) & (y0 <= _H - 2)).astype(jnp.float32)
    wl = wx0 * inx + wx1 * (x0 == -1)
    wr = wx1 * inx + wx0 * (x0 == _W - 1)
    wt = wy0 * iny + wy1 * (y0 == -1)
    wb = wy1 * iny + wy0 * (y0 == _H - 1)
    xb = jnp.clip(x0, 0, _W - 2)
    yb = jnp.clip(y0, 0, _H - 2)
    iv_f = lax.bitcast_convert_type((yb << 8) | xb, jnp.float32)
    rows = [iv_f, wt * wl, wt * wr, wb * wl, wb * wr]
    stacked = jnp.stack(
        [r.reshape(_N, _NCHUNK, _RC, _W) for r in rows], axis=2
    ).reshape(_N, _NCHUNK, 5 * _RC, _W)
    padded = jnp.pad(
        stacked, ((0, 0), (0, 0), (0, _CROWS - 5 * _RC), (0, 256 - _W))
    )
    iw_ref[...] = padded.reshape(_N, _NCHUNK * _CROWS, 256)


def _prep(gx, gy):
    return pl.pallas_call(
        _prep_body,
        out_shape=jax.ShapeDtypeStruct((_N, _NCHUNK * _CROWS, 256), jnp.float32),
    )(gx, gy)


_mesh = plsc.VectorSubcoreMesh(core_axis_name="c", subcore_axis_name="s")


@functools.partial(
    pl.kernel,
    out_type=jax.ShapeDtypeStruct((_N, _C, _H, _W), jnp.float32),
    mesh=_mesh,
    compiler_params=pltpu.CompilerParams(needs_layout_passes=False),
    scratch_types=[
        pltpu.VMEM((2 * _H, 128), jnp.int32),
        pltpu.VMEM((2 * _H, 128), jnp.int32),
        (pltpu.VMEM((_CROWS, 256), jnp.float32),
         pltpu.VMEM((_CROWS, 256), jnp.float32)),
        pltpu.VMEM((2, 4, _RC, _W), jnp.float32),
        pltpu.SemaphoreType.DMA,
        (pltpu.SemaphoreType.DMA, pltpu.SemaphoreType.DMA),
        (pltpu.SemaphoreType.DMA, pltpu.SemaphoreType.DMA),
    ],
)
def _sc_sample(
    xp_hbm, iw_hbm, out_hbm,
    pp0_v, pp1_v, iw_v, out_v, sem_pl, sem_iw, sem_out,
):
    wid = lax.axis_index("s") * _NC + lax.axis_index("c")
    base_pp = wid * _PPW
    b = base_pp // _CH
    base_q = base_pp - b * _CH

    def iw_copy(c, buf):
        return pltpu.make_async_copy(
            iw_hbm.at[b, pl.ds(c * _CROWS, _CROWS)], iw_v[buf], sem_iw[buf]
        )

    def out_copy(buf, ch, c):
        return pltpu.make_async_copy(
            out_v.at[buf],
            out_hbm.at[b, pl.ds(ch, 4), pl.ds(c * _RC, _RC)],
            sem_out[buf],
        )

    def pair_body(pp, _):
        q0 = base_pp + 2 * pp
        ch0 = 2 * (base_q + 2 * pp)
        cp0 = pltpu.async_copy(xp_hbm.at[q0], pp0_v, sem_pl)
        cp1 = pltpu.async_copy(xp_hbm.at[q0 + 1], pp1_v, sem_pl)
        iw_copy(0, 0).start()
        iw_copy(1, 1).start()
        cp0.wait()
        cp1.wait()

        def chunk2_body(cc, _):
            for buf in (0, 1):
                c = cc * 2 + buf
                iw_copy(c, buf).wait()

                @pl.when(c >= 2)
                def _():
                    out_copy(buf, ch0, c).wait()

                iwb = iw_v[buf]

                @plsc.parallel_loop(0, _W, step=16, unroll=1)
                def vec_body(x0):
                    for r in range(_RC):
                        iv = plsc.bitcast(iwb[r, pl.ds(x0, 16)], jnp.int32)
                        w00 = iwb[_RC + r, pl.ds(x0, 16)]
                        w01 = iwb[2 * _RC + r, pl.ds(x0, 16)]
                        w10 = iwb[3 * _RC + r, pl.ds(x0, 16)]
                        w11 = iwb[4 * _RC + r, pl.ds(x0, 16)]
                        iv1 = iv + 1
                        iv2 = iv + 256
                        iv3 = iv + 257
                        taps = [(t >> 7, t & 127) for t in (iv, iv1, iv2, iv3)]
                        for k, ppv in ((0, pp0_v), (1, pp1_v)):
                            g00 = plsc.load_gather(ppv, list(taps[0]))
                            g01 = plsc.load_gather(ppv, list(taps[1]))
                            g10 = plsc.load_gather(ppv, list(taps[2]))
                            g11 = plsc.load_gather(ppv, list(taps[3]))
                            acc_a = (
                                plsc.bitcast(g00 & _MASK_HI, jnp.float32) * w00
                                + plsc.bitcast(g01 & _MASK_HI, jnp.float32) * w01
                                + plsc.bitcast(g10 & _MASK_HI, jnp.float32) * w10
                                + plsc.bitcast(g11 & _MASK_HI, jnp.float32) * w11
                            )
                            acc_b = (
                                plsc.bitcast(g00 << 16, jnp.float32) * w00
                                + plsc.bitcast(g01 << 16, jnp.float32) * w01
                                + plsc.bitcast(g10 << 16, jnp.float32) * w10
                                + plsc.bitcast(g11 << 16, jnp.float32) * w11
                            )
                            out_v[buf, 2 * k, r, pl.ds(x0, 16)] = acc_a
                            out_v[buf, 2 * k + 1, r, pl.ds(x0, 16)] = acc_b

                @pl.when(c + 2 < _NCHUNK)
                def _():
                    iw_copy(c + 2, buf).start()

                out_copy(buf, ch0, c).start()
            return 0

        lax.fori_loop(0, _NCHUNK // 2, chunk2_body, 0)
        for buf in (0, 1):
            out_copy(buf, ch0, 0).wait()
        return 0

    lax.fori_loop(0, _PPW // 2, pair_body, 0)


def kernel(x, grid):
    xp = _pack(x)
    gx = grid[..., 0].reshape(_N, _HW)
    gy = grid[..., 1].reshape(_N, _HW)
    iw = _prep(gx, gy)
    return _sc_sample(xp, iw)
